# baseline (speedup 1.0000x reference)
import functools

import jax
import jax.numpy as jnp
from jax import lax
from jax.experimental import pallas as pl
from jax.experimental.pallas import tpu as pltpu
from jax.experimental.pallas import tpu_sc as plsc

_NUM_CORES = 2
_NUM_SUBCORES = 16
_NUM_WORKERS = _NUM_CORES * _NUM_SUBCORES
_LANES = 16


_ROWS_PER_GROUP = 8


def _sc_pool_fn(batch, ctx, vocab, embed):
  bpw = batch // _NUM_WORKERS
  rpg = _ROWS_PER_GROUP
  gpw = bpw // rpg
  ipg = rpg * ctx
  lanes_per_row = embed // _LANES
  inv_ctx = 1.0 / ctx

  def body(table_hbm, ids_hbm, out_hbm, raw_v, rows_a, rows_b, acc_v,
           sem_a, sem_b):
    wid = lax.axis_index("s") * _NUM_CORES + lax.axis_index("c")
    pltpu.sync_copy(ids_hbm.at[pl.ds(wid * gpw, gpw)], raw_v)

    bufs = (rows_a, rows_b)
    sems = (sem_a, sem_b)

    def fire(g, b):
      return pltpu.async_copy(table_hbm.at[raw_v.at[g]], bufs[b], sems[b])

    def drain(b):
      pltpu.make_async_copy(
          table_hbm.at[pl.ds(0, ipg)], bufs[b], sems[b]).wait()

    fire(0, 0)
    fire(1, 1)

    def pair_body(i, carry):
      for b in range(2):
        g = 2 * i + b
        drain(b)
        buf = bufs[b]
        for j in range(rpg):
          for cc in range(lanes_per_row):
            s = buf[j * ctx, pl.ds(cc * _LANES, _LANES)]
            for t in range(1, ctx):
              s = s + buf[j * ctx + t, pl.ds(cc * _LANES, _LANES)]
            acc_v[g * rpg + j, pl.ds(cc * _LANES, _LANES)] = s * inv_ctx

        @pl.when(g + 2 < gpw)
        def _():
          fire(g + 2, b)
      return carry

    lax.fori_loop(0, gpw // 2, pair_body, 0)
    pltpu.sync_copy(acc_v, out_hbm.at[pl.ds(wid * bpw, bpw)])

  return pl.kernel(
      body,
      out_type=jax.ShapeDtypeStruct((batch, embed), jnp.float32),
      mesh=plsc.VectorSubcoreMesh(core_axis_name="c", subcore_axis_name="s"),
      compiler_params=pltpu.CompilerParams(use_tc_tiling_on_sc=False),
      scratch_types=[
          pltpu.VMEM((gpw, ipg), jnp.int32),
          pltpu.VMEM((ipg, embed), jnp.float32),
          pltpu.VMEM((ipg, embed), jnp.float32),
          pltpu.VMEM((bpw, embed), jnp.float32),
          pltpu.SemaphoreType.DMA,
          pltpu.SemaphoreType.DMA,
      ],
  )


def _mm_body(p_ref, wt_ref, b_ref, o_ref):
  o_ref[...] = lax.dot_general(
      wt_ref[...], p_ref[...],
      dimension_numbers=(((0,), (1,)), ((), ())),
      preferred_element_type=jnp.float32) + b_ref[...][:, None]


def _mm_fn(batch, vocab, embed, n_tile):
  grid = (pl.cdiv(vocab, n_tile),)
  return pl.pallas_call(
      _mm_body,
      grid=grid,
      in_specs=[
          pl.BlockSpec((batch, embed), lambda i: (0, 0)),
          pl.BlockSpec((embed, n_tile), lambda i: (0, i)),
          pl.BlockSpec((n_tile,), lambda i: (i,)),
      ],
      out_specs=pl.BlockSpec((n_tile, batch), lambda i: (i, 0)),
      out_shape=jax.ShapeDtypeStruct((vocab, batch), jnp.float32),
  )


@functools.lru_cache(maxsize=None)
def _build(batch, ctx, vocab, embed):
  return _sc_pool_fn(batch, ctx, vocab, embed), _mm_fn(batch, vocab, embed, 512)


def kernel(context_ids, embed_table, fc_weight, fc_bias):
  batch, ctx = context_ids.shape
  vocab, embed = embed_table.shape
  sc_pool, mm = _build(batch, ctx, vocab, embed)
  ids = context_ids.astype(jnp.int32).reshape(
      batch // _ROWS_PER_GROUP, _ROWS_PER_GROUP * ctx)
  pooled = sc_pool(embed_table, ids)
  return mm(pooled, fc_weight.T, fc_bias).T
